# baseline (speedup 1.0000x reference)
import functools

import jax
import jax.numpy as jnp
import numpy as np
from jax import lax
from jax.experimental import pallas as pl
from jax.experimental.pallas import tpu as pltpu
from jax.experimental.pallas import tpu_sc as plsc

_NUM_FIELDS = 26
_EMBED_DIM = 16
_BATCH = 16384
_FIELD_SIZE = 100000
_TOTAL = _NUM_FIELDS * _FIELD_SIZE
_LIN_PAD = 64
_LIN_ROWS = (_TOTAL + _LIN_PAD) // 128

_NC = 2
_NS = 16
_NW = _NC * _NS
_SPW = _BATCH // _NW
_CHUNK = 16
_NCHUNKS = _SPW // _CHUNK
_ROWS = _CHUNK * _NUM_FIELDS


def _fm_body(x_hbm, emb_hbm, lin_hbm, out_hbm,
             x_v, eidx_v, lidx_v, ecol_v, lcol_v,
             rows_v, lin_v, out_v, sem_e, sem_l):
  wid = lax.axis_index("s") * _NC + lax.axis_index("c")
  lane = lax.iota(jnp.int32, 16)
  off_lo = lane * _FIELD_SIZE
  off_hi = (lane + 10) * _FIELD_SIZE

  def chunk_body(c, _):
    base = wid * _SPW + c * _CHUNK
    pltpu.sync_copy(x_hbm.at[pl.ds(base, _CHUNK)], x_v)

    def idx_body(i, _):
      lo = x_v[i, pl.ds(0, 16)] + off_lo
      hi = x_v[i, pl.ds(10, 16)] + off_hi
      k0 = i * _NUM_FIELDS
      eidx_v[pl.ds(k0, 16)] = lo >> 3
      eidx_v[pl.ds(k0 + 10, 16)] = hi >> 3
      lidx_v[pl.ds(k0, 16)] = lo >> 7
      lidx_v[pl.ds(k0 + 10, 16)] = hi >> 7
      ecol_v[pl.ds(k0, 16)] = (lo & 7) * 16
      ecol_v[pl.ds(k0 + 10, 16)] = (hi & 7) * 16
      lcol_v[pl.ds(k0, 16)] = lo & 127
      lcol_v[pl.ds(k0 + 10, 16)] = hi & 127
      return _
    lax.fori_loop(0, _CHUNK, idx_body, None)

    cp_e = pltpu.make_async_copy(emb_hbm.at[eidx_v], rows_v, sem_e)
    cp_l = pltpu.make_async_copy(lin_hbm.at[lidx_v], lin_v, sem_l)
    cp_e.start()
    cp_l.start()
    cp_e.wait()
    cp_l.wait()

    rowb = lane * _NUM_FIELDS
    acc = jnp.zeros((16,), jnp.float32)
    for f in range(_NUM_FIELDS):
      rpos = rowb + f
      lc = plsc.load_gather(lcol_v, [rpos])
      acc = acc + plsc.load_gather(lin_v, [rpos, lc])
    total = jnp.zeros((16,), jnp.float32)
    for d in range(_EMBED_DIM):
      rpos = rowb
      cb = plsc.load_gather(ecol_v, [rpos])
      v = plsc.load_gather(rows_v, [rpos, cb + d])
      s = v
      ss = v * v
      for f in range(1, _NUM_FIELDS):
        rpos = rowb + f
        cb = plsc.load_gather(ecol_v, [rpos])
        v = plsc.load_gather(rows_v, [rpos, cb + d])
        s = s + v
        ss = ss + v * v
      total = total + (s * s - ss)
    out_v[...] = 0.5 * total + acc

    pltpu.sync_copy(out_v, out_hbm.at[pl.ds(base, _CHUNK)])
    return _

  lax.fori_loop(0, _NCHUNKS, chunk_body, None)


@jax.jit
def _fm(x, emb128, lin128):
  mesh = plsc.VectorSubcoreMesh(core_axis_name="c", subcore_axis_name="s",
                                num_cores=_NC, num_subcores=_NS)
  f = pl.kernel(
      _fm_body,
      out_type=jax.ShapeDtypeStruct((_BATCH,), jnp.float32),
      mesh=mesh,
      compiler_params=pltpu.CompilerParams(needs_layout_passes=False,
                                           use_tc_tiling_on_sc=True),
      scratch_types=[
          pltpu.VMEM((_CHUNK, _NUM_FIELDS), jnp.int32),
          pltpu.VMEM((_ROWS,), jnp.int32),
          pltpu.VMEM((_ROWS,), jnp.int32),
          pltpu.VMEM((_ROWS,), jnp.int32),
          pltpu.VMEM((_ROWS,), jnp.int32),
          pltpu.VMEM((_ROWS, 128), jnp.float32),
          pltpu.VMEM((_ROWS, 128), jnp.float32),
          pltpu.VMEM((_CHUNK,), jnp.float32),
          pltpu.SemaphoreType.DMA,
          pltpu.SemaphoreType.DMA,
      ],
  )
  return f(x, emb128, lin128)


def kernel(x, emb_table, lin_weight, bias):
  emb128 = emb_table.reshape(_TOTAL // 8, 128)
  lin128 = jnp.pad(lin_weight[:, 0], (0, _LIN_PAD)).reshape(_LIN_ROWS, 128)
  return _fm(x, emb128, lin128) + bias[0]


# speedup vs baseline: 1.1242x; 1.1242x over previous
import functools

import jax
import jax.numpy as jnp
import numpy as np
from jax import lax
from jax.experimental import pallas as pl
from jax.experimental.pallas import tpu as pltpu
from jax.experimental.pallas import tpu_sc as plsc

_NUM_FIELDS = 26
_EMBED_DIM = 16
_BATCH = 16384
_FIELD_SIZE = 100000
_TOTAL = _NUM_FIELDS * _FIELD_SIZE

_NC = 2
_NS = 16
_NW = _NC * _NS
_SPW = _BATCH // _NW
_CHUNK = 128
_NCHUNKS = _SPW // _CHUNK
_ROWS = _CHUNK * _NUM_FIELDS


def _fm_body(x_hbm, emb_hbm, lin_hbm, out_hbm,
             x_v, idx_v, idx2_v, rows_v, lin_v, out_v, sem_e, sem_l):
  wid = lax.axis_index("s") * _NC + lax.axis_index("c")
  lane = lax.iota(jnp.int32, 16)
  off_lo = lane * _FIELD_SIZE
  off_hi = (lane + 10) * _FIELD_SIZE

  def chunk_body(c, _):
    base = wid * _SPW + c * _CHUNK
    pltpu.sync_copy(x_hbm.at[pl.ds(base, _CHUNK)], x_v)

    def idx_body(i, _):
      lo = x_v[i, pl.ds(0, 16)] + off_lo
      hi = x_v[i, pl.ds(10, 16)] + off_hi
      idx_v[pl.ds(i * _NUM_FIELDS, 16)] = lo
      idx_v[pl.ds(i * _NUM_FIELDS + 10, 16)] = hi
      idx2_v[pl.ds(i * _NUM_FIELDS, 16)] = lo >> 4
      idx2_v[pl.ds(i * _NUM_FIELDS + 10, 16)] = hi >> 4
      return _
    lax.fori_loop(0, _CHUNK, idx_body, None)

    cp_e = pltpu.make_async_copy(emb_hbm.at[idx_v], rows_v, sem_e)
    cp_l = pltpu.make_async_copy(lin_hbm.at[idx2_v], lin_v, sem_l)
    cp_e.start()
    cp_l.start()
    cp_e.wait()
    cp_l.wait()

    def group_body(g, _):
      rowb = lane * _NUM_FIELDS + g * (16 * _NUM_FIELDS)
      acc = jnp.zeros((16,), jnp.float32)
      for f in range(_NUM_FIELDS):
        gidx = plsc.load_gather(idx_v, [rowb + f])
        acc = acc + plsc.load_gather(lin_v, [rowb + f, gidx & 15])
      total = jnp.zeros((16,), jnp.float32)
      for d in range(_EMBED_DIM):
        dsplat = jnp.full((16,), d, jnp.int32)
        v = plsc.load_gather(rows_v, [rowb, dsplat])
        s = v
        ss = v * v
        for f in range(1, _NUM_FIELDS):
          v = plsc.load_gather(rows_v, [rowb + f, dsplat])
          s = s + v
          ss = ss + v * v
        total = total + (s * s - ss)
      out_v[pl.ds(g * 16, 16)] = 0.5 * total + acc
      return _
    lax.fori_loop(0, _CHUNK // 16, group_body, None)

    pltpu.sync_copy(out_v, out_hbm.at[pl.ds(base, _CHUNK)])
    return _

  lax.fori_loop(0, _NCHUNKS, chunk_body, None)


@jax.jit
def _fm(x, emb_table, lin16):
  mesh = plsc.VectorSubcoreMesh(core_axis_name="c", subcore_axis_name="s",
                                num_cores=_NC, num_subcores=_NS)
  f = pl.kernel(
      _fm_body,
      out_type=jax.ShapeDtypeStruct((_BATCH,), jnp.float32),
      mesh=mesh,
      compiler_params=pltpu.CompilerParams(needs_layout_passes=False,
                                           use_tc_tiling_on_sc=False),
      scratch_types=[
          pltpu.VMEM((_CHUNK, _NUM_FIELDS), jnp.int32),
          pltpu.VMEM((_ROWS,), jnp.int32),
          pltpu.VMEM((_ROWS,), jnp.int32),
          pltpu.VMEM((_ROWS, _EMBED_DIM), jnp.float32),
          pltpu.VMEM((_ROWS, 16), jnp.float32),
          pltpu.VMEM((_CHUNK,), jnp.float32),
          pltpu.SemaphoreType.DMA,
          pltpu.SemaphoreType.DMA,
      ],
  )
  return f(x, emb_table, lin16)


def kernel(x, emb_table, lin_weight, bias):
  return _fm(x, emb_table, lin_weight.reshape(_TOTAL // 16, 16)) + bias[0]


# speedup vs baseline: 1.1300x; 1.0051x over previous
import jax
import jax.numpy as jnp
from jax import lax
from jax.experimental import pallas as pl
from jax.experimental.pallas import tpu as pltpu
from jax.experimental.pallas import tpu_sc as plsc

_NUM_FIELDS = 26
_EMBED_DIM = 16
_BATCH = 16384
_FIELD_SIZE = 100000
_TOTAL = _NUM_FIELDS * _FIELD_SIZE

_NC = 2
_NS = 16
_NW = _NC * _NS
_SPW = _BATCH // _NW
_CHUNK = 128
_NCHUNKS = _SPW // _CHUNK
_ROWS = _CHUNK * _NUM_FIELDS


def _fm_body(x_hbm, emb_hbm, lin_hbm, out_hbm,
             x_v, idx_v, idx2_v, rows_v, lin_v, out_v, sem_e, sem_l):
  wid = lax.axis_index("s") * _NC + lax.axis_index("c")
  lane = lax.iota(jnp.int32, 16)
  off_lo = lane * _FIELD_SIZE
  off_hi = (lane + 10) * _FIELD_SIZE

  def chunk_body(c, _):
    base = wid * _SPW + c * _CHUNK
    pltpu.sync_copy(x_hbm.at[pl.ds(base, _CHUNK)], x_v)

    def idx_body(i, _):
      lo = x_v[i, pl.ds(0, 16)] + off_lo
      hi = x_v[i, pl.ds(10, 16)] + off_hi
      idx_v[pl.ds(i * _NUM_FIELDS, 16)] = lo
      idx_v[pl.ds(i * _NUM_FIELDS + 10, 16)] = hi
      idx2_v[pl.ds(i * _NUM_FIELDS, 16)] = lo >> 4
      idx2_v[pl.ds(i * _NUM_FIELDS + 10, 16)] = hi >> 4
      return _
    lax.fori_loop(0, _CHUNK, idx_body, None)

    cp_e = pltpu.make_async_copy(emb_hbm.at[idx_v], rows_v, sem_e)
    cp_l = pltpu.make_async_copy(lin_hbm.at[idx2_v], lin_v, sem_l)
    cp_e.start()
    cp_l.start()
    cp_e.wait()
    cp_l.wait()

    def group_body(g, _):
      rowb = lane * _NUM_FIELDS + g * (16 * _NUM_FIELDS)
      acc = jnp.zeros((16,), jnp.float32)
      for f in range(_NUM_FIELDS):
        gidx = plsc.load_gather(idx_v, [rowb + f])
        acc = acc + plsc.load_gather(lin_v, [rowb + f, gidx & 15])
      total = jnp.zeros((16,), jnp.float32)
      for d in range(_EMBED_DIM):
        dsplat = jnp.full((16,), d, jnp.int32)
        v = plsc.load_gather(rows_v, [rowb, dsplat])
        s = v
        ss = v * v
        for f in range(1, _NUM_FIELDS):
          v = plsc.load_gather(rows_v, [rowb + f, dsplat])
          s = s + v
          ss = ss + v * v
        total = total + (s * s - ss)
      out_v[pl.ds(g * 16, 16)] = 0.5 * total + acc
      return _
    lax.fori_loop(0, _CHUNK // 16, group_body, None)

    pltpu.sync_copy(out_v, out_hbm.at[pl.ds(base, _CHUNK)])
    return _

  lax.fori_loop(0, _NCHUNKS, chunk_body, None)


@jax.jit
def _fm(x, emb_table, lin16):
  mesh = plsc.VectorSubcoreMesh(core_axis_name="c", subcore_axis_name="s",
                                num_cores=_NC, num_subcores=_NS)
  f = pl.kernel(
      _fm_body,
      out_type=jax.ShapeDtypeStruct((_BATCH,), jnp.float32),
      mesh=mesh,
      compiler_params=pltpu.CompilerParams(needs_layout_passes=False,
                                           use_tc_tiling_on_sc=False),
      scratch_types=[
          pltpu.VMEM((_CHUNK, _NUM_FIELDS), jnp.int32),
          pltpu.VMEM((_ROWS,), jnp.int32),
          pltpu.VMEM((_ROWS,), jnp.int32),
          pltpu.VMEM((_ROWS, _EMBED_DIM), jnp.float32),
          pltpu.VMEM((_ROWS, 16), jnp.float32),
          pltpu.VMEM((_CHUNK,), jnp.float32),
          pltpu.SemaphoreType.DMA,
          pltpu.SemaphoreType.DMA,
      ],
  )
  return f(x, emb_table, lin16)


def kernel(x, emb_table, lin_weight, bias):
  return _fm(x, emb_table, lin_weight.reshape(_TOTAL // 16, 16)) + bias[0]
